# baseline (speedup 1.0000x reference)
import functools

import jax
import jax.numpy as jnp
from jax import lax
from jax.experimental import pallas as pl
from jax.experimental.pallas import tpu as pltpu
from jax.experimental.pallas import tpu_sc as plsc

N_IDROWS = 16384
PIECE_IDROWS = N_IDROWS // 4
ROW_W = 50
EMBED_D = 32
NUM_WORKERS = 32
IDROWS_PER_W = PIECE_IDROWS // NUM_WORKERS
CHUNK = 8
NCHUNKS = IDROWS_PER_W // CHUNK


def _gather_body(piece, idx_hbm, table_hbm, out_hbm, idx_v, rows_a, rows_b,
                 sem_ga, sem_gb, sem_oa, sem_ob):
    wid = lax.axis_index("s") * 2 + lax.axis_index("c")
    row0 = wid * IDROWS_PER_W
    pltpu.sync_copy(
        idx_hbm.at[pl.ds(piece * PIECE_IDROWS + row0, IDROWS_PER_W)], idx_v)

    def fire_gathers(c, buf, sem):
        for g in range(CHUNK):
            pltpu.async_copy(
                table_hbm.at[idx_v.at[c * CHUNK + g]],
                buf.at[g],
                sem,
            )

    def drain_gathers(buf, sem):
        pltpu.make_async_copy(out_hbm.at[pl.ds(0, CHUNK)], buf, sem).wait()

    def fire_out(c, buf, sem):
        pltpu.async_copy(buf, out_hbm.at[pl.ds(row0 + c * CHUNK, CHUNK)], sem)

    def drain_out(c, buf, sem):
        pltpu.make_async_copy(
            buf, out_hbm.at[pl.ds(row0 + c * CHUNK, CHUNK)], sem).wait()

    fire_gathers(0, rows_a, sem_ga)
    fire_gathers(1, rows_b, sem_gb)

    def group_step(g, _):
        c = 2 * g
        drain_gathers(rows_a, sem_ga)
        fire_out(c, rows_a, sem_oa)
        drain_gathers(rows_b, sem_gb)
        fire_out(c + 1, rows_b, sem_ob)
        drain_out(c, rows_a, sem_oa)
        fire_gathers(c + 2, rows_a, sem_ga)
        drain_out(c + 1, rows_b, sem_ob)
        fire_gathers(c + 3, rows_b, sem_gb)
        return 0

    lax.fori_loop(0, NCHUNKS // 2 - 1, group_step, 0)

    c = NCHUNKS - 2
    drain_gathers(rows_a, sem_ga)
    fire_out(c, rows_a, sem_oa)
    drain_gathers(rows_b, sem_gb)
    fire_out(c + 1, rows_b, sem_ob)
    drain_out(c, rows_a, sem_oa)
    drain_out(c + 1, rows_b, sem_ob)


def _make_gather(piece):
  return functools.partial(
    pl.kernel,
    mesh=plsc.VectorSubcoreMesh(core_axis_name="c", subcore_axis_name="s"),
    out_type=jax.ShapeDtypeStruct((PIECE_IDROWS, ROW_W, EMBED_D), jnp.float32),
    scratch_types=[
        pltpu.VMEM((IDROWS_PER_W, ROW_W), jnp.int32),
        pltpu.VMEM((CHUNK, ROW_W, EMBED_D), jnp.float32),
        pltpu.VMEM((CHUNK, ROW_W, EMBED_D), jnp.float32),
        pltpu.SemaphoreType.DMA,
        pltpu.SemaphoreType.DMA,
        pltpu.SemaphoreType.DMA,
        pltpu.SemaphoreType.DMA,
    ],
    compiler_params=pltpu.CompilerParams(use_tc_tiling_on_sc=False),
  )(functools.partial(_gather_body, piece))


_gathers = [_make_gather(k) for k in range(4)]


def kernel(ids, embeddings):
    pieces = [g(ids, embeddings) for g in _gathers]
    return jnp.concatenate(pieces, axis=0)


# speedup vs baseline: 1.0045x; 1.0045x over previous
import functools

import jax
import jax.numpy as jnp
from jax import lax
from jax.experimental import pallas as pl
from jax.experimental.pallas import tpu as pltpu
from jax.experimental.pallas import tpu_sc as plsc

N_IDROWS = 16384
PIECE_IDROWS = N_IDROWS // 4
ROW_W = 50
EMBED_D = 32
NUM_WORKERS = 32
IDROWS_PER_W = PIECE_IDROWS // NUM_WORKERS
CHUNK = 8
NCHUNKS = IDROWS_PER_W // CHUNK


def _gather_body(idx_hbm, table_hbm, out_hbm, idx_v, rows_a, rows_b, sem_ga,
                 sem_gb, sem_oa, sem_ob):
    wid = lax.axis_index("s") * 2 + lax.axis_index("c")
    row0 = wid * IDROWS_PER_W
    pltpu.sync_copy(idx_hbm.at[pl.ds(row0, IDROWS_PER_W)], idx_v)

    def fire_gathers(c, buf, sem):
        for g in range(CHUNK):
            pltpu.async_copy(
                table_hbm.at[idx_v.at[c * CHUNK + g]],
                buf.at[g],
                sem,
            )

    def drain_gathers(buf, sem):
        pltpu.make_async_copy(out_hbm.at[pl.ds(0, CHUNK)], buf, sem).wait()

    def fire_out(c, buf, sem):
        pltpu.async_copy(buf, out_hbm.at[pl.ds(row0 + c * CHUNK, CHUNK)], sem)

    def drain_out(c, buf, sem):
        pltpu.make_async_copy(
            buf, out_hbm.at[pl.ds(row0 + c * CHUNK, CHUNK)], sem).wait()

    fire_gathers(0, rows_a, sem_ga)
    fire_gathers(1, rows_b, sem_gb)

    def group_step(g, _):
        c = 2 * g
        drain_gathers(rows_a, sem_ga)
        fire_out(c, rows_a, sem_oa)
        drain_gathers(rows_b, sem_gb)
        fire_out(c + 1, rows_b, sem_ob)
        drain_out(c, rows_a, sem_oa)
        fire_gathers(c + 2, rows_a, sem_ga)
        drain_out(c + 1, rows_b, sem_ob)
        fire_gathers(c + 3, rows_b, sem_gb)
        return 0

    lax.fori_loop(0, NCHUNKS // 2 - 1, group_step, 0)

    c = NCHUNKS - 2
    drain_gathers(rows_a, sem_ga)
    fire_out(c, rows_a, sem_oa)
    drain_gathers(rows_b, sem_gb)
    fire_out(c + 1, rows_b, sem_ob)
    drain_out(c, rows_a, sem_oa)
    drain_out(c + 1, rows_b, sem_ob)


_gather = functools.partial(
    pl.kernel,
    mesh=plsc.VectorSubcoreMesh(core_axis_name="c", subcore_axis_name="s"),
    out_type=jax.ShapeDtypeStruct((PIECE_IDROWS, ROW_W, EMBED_D), jnp.float32),
    scratch_types=[
        pltpu.VMEM((IDROWS_PER_W, ROW_W), jnp.int32),
        pltpu.VMEM((CHUNK, ROW_W, EMBED_D), jnp.float32),
        pltpu.VMEM((CHUNK, ROW_W, EMBED_D), jnp.float32),
        pltpu.SemaphoreType.DMA,
        pltpu.SemaphoreType.DMA,
        pltpu.SemaphoreType.DMA,
        pltpu.SemaphoreType.DMA,
    ],
    compiler_params=pltpu.CompilerParams(use_tc_tiling_on_sc=False),
)(_gather_body)


def kernel(ids, embeddings):
    pieces = [
        _gather(ids[k * PIECE_IDROWS:(k + 1) * PIECE_IDROWS], embeddings)
        for k in range(4)
    ]
    return jnp.concatenate(pieces, axis=0)
